# Initial kernel scaffold; baseline (speedup 1.0000x reference)
import jax
import jax.numpy as jnp
from jax.experimental import pallas as pl


def kernel(descriptors0, descriptors1, keypoints0, keypoints1):
    raise NotImplementedError("write your pallas kernel here")


# speedup vs baseline: 1.8161x; 1.8161x over previous
import functools

import jax
import jax.numpy as jnp
from jax import lax
from jax.experimental import pallas as pl
from jax.experimental.pallas import tpu as pltpu
from jax.experimental.pallas import tpu_sc as plsc

N1 = 4096
N2 = 4096
D = 256
BM = 256

NC = 2
NS = 16
NW = NC * NS
CHUNK = N1 // NW
L = 16

def _stage1_body(d0_ref, d1_ref, nn12_ref, sc_ref, nn21_ref, cmax_ref):
    i = pl.program_id(0)
    sim = jax.lax.dot_general(
        d0_ref[...], d1_ref[...],
        dimension_numbers=(((0,), (0,)), ((), ())),
        preferred_element_type=jnp.float32,
    )

    rmax = jnp.max(sim, axis=1, keepdims=True)
    lane_ids = lax.broadcasted_iota(jnp.int32, sim.shape, 1)
    rarg = jnp.min(jnp.where(sim == rmax, lane_ids, 2**30),
                   axis=1, keepdims=True)
    nn12_ref[...] = rarg
    sc_ref[...] = rmax

    bmax = jnp.max(sim, axis=0, keepdims=True)
    row_ids = lax.broadcasted_iota(jnp.int32, sim.shape, 0) + i * BM
    barg = jnp.min(jnp.where(sim == bmax, row_ids, 2**30),
                   axis=0, keepdims=True)

    @pl.when(i == 0)
    def _init():
        cmax_ref[...] = bmax
        nn21_ref[...] = barg

    @pl.when(i > 0)
    def _update():
        prev_max = cmax_ref[...]
        prev_arg = nn21_ref[...]
        better = bmax > prev_max
        cmax_ref[...] = jnp.where(better, bmax, prev_max)
        nn21_ref[...] = jnp.where(better, barg, prev_arg)


def _stage1(d0, d1):
    grid = (N1 // BM,)
    return pl.pallas_call(
        _stage1_body,
        grid=grid,
        in_specs=[
            pl.BlockSpec((D, BM), lambda i: (0, i)),
            pl.BlockSpec((D, N2), lambda i: (0, 0)),
        ],
        out_specs=[
            pl.BlockSpec((BM, 1), lambda i: (i, 0)),
            pl.BlockSpec((BM, 1), lambda i: (i, 0)),
            pl.BlockSpec((1, N2), lambda i: (0, 0)),
        ],
        out_shape=[
            jax.ShapeDtypeStruct((N1, 1), jnp.int32),
            jax.ShapeDtypeStruct((N1, 1), jnp.float32),
            jax.ShapeDtypeStruct((1, N2), jnp.int32),
        ],
        scratch_shapes=[pltpu.VMEM((1, N2), jnp.float32)],
    )(d0, d1)


def _stage2_body(nn12_hbm, nn21_hbm, sc_hbm, oi_hbm, os_hbm,
                 idx_v, g_v, s_v, oi_v, os_v, sem):
    c = lax.axis_index("c")
    s = lax.axis_index("s")
    wid = s * NC + c
    base = wid * CHUNK
    pltpu.sync_copy(nn12_hbm.at[pl.ds(base, CHUNK)], idx_v)
    pltpu.sync_copy(sc_hbm.at[pl.ds(base, CHUNK)], s_v)
    pltpu.async_copy(nn21_hbm.at[idx_v], g_v, sem).wait()
    for k in range(CHUNK // L):
        idx = idx_v[pl.ds(k * L, L)]
        g = g_v[pl.ds(k * L, L)]
        ids = lax.iota(jnp.int32, L) + (base + k * L)
        m = g == ids
        oi_v[pl.ds(k * L, L)] = jnp.where(m, idx, jnp.int32(-1))
        os_v[pl.ds(k * L, L)] = jnp.where(m, s_v[pl.ds(k * L, L)],
                                          jnp.float32(-1.0))
    pltpu.sync_copy(oi_v, oi_hbm.at[pl.ds(base, CHUNK)])
    pltpu.sync_copy(os_v, os_hbm.at[pl.ds(base, CHUNK)])


@functools.lru_cache(maxsize=1)
def _get_stage2():
    return functools.partial(
        pl.kernel,
        mesh=plsc.VectorSubcoreMesh(core_axis_name="c", subcore_axis_name="s"),
        out_type=[
            jax.ShapeDtypeStruct((N1,), jnp.int32),
            jax.ShapeDtypeStruct((N1,), jnp.float32),
        ],
        scratch_types=[
            pltpu.VMEM((CHUNK,), jnp.int32),
            pltpu.VMEM((CHUNK,), jnp.int32),
            pltpu.VMEM((CHUNK,), jnp.float32),
            pltpu.VMEM((CHUNK,), jnp.int32),
            pltpu.VMEM((CHUNK,), jnp.float32),
            pltpu.SemaphoreType.DMA,
        ],
    )(_stage2_body)


def kernel(descriptors0, descriptors1, keypoints0, keypoints1):
    d0 = jnp.squeeze(descriptors0, axis=0)
    d1 = jnp.squeeze(descriptors1, axis=0)
    nn12, scores, nn21 = _stage1(d0, d1)
    oi, os = _get_stage2()(nn12.reshape(N1), nn21.reshape(N2),
                           scores.reshape(N1))
    indices0 = oi[None, :]
    mscores0 = os[None, :]
    return indices0, indices0, mscores0, mscores0


# speedup vs baseline: 2.1040x; 1.1586x over previous
import functools

import jax
import jax.numpy as jnp
from jax import lax
from jax.experimental import pallas as pl
from jax.experimental.pallas import tpu as pltpu
from jax.experimental.pallas import tpu_sc as plsc

N1 = 4096
N2 = 4096
D = 256
BM = 256

NC = 2
NS = 16
NW = NC * NS
CHUNK = N1 // NW
L = 16

def _stage1_body(d0_ref, d1_ref, nn12_ref, sc_ref, nn21_ref, cmax_ref):
    i = pl.program_id(0)
    sim = jax.lax.dot_general(
        d0_ref[...], d1_ref[...],
        dimension_numbers=(((0,), (0,)), ((), ())),
        preferred_element_type=jnp.float32,
    )

    LC = 128
    rv = sim[:, 0:LC]
    rc = jnp.zeros((BM, LC), jnp.int32)
    for c in range(1, N2 // LC):
        v = sim[:, c * LC:(c + 1) * LC]
        m = v > rv
        rv = jnp.where(m, v, rv)
        rc = jnp.where(m, c, rc)
    rmax = jnp.max(rv, axis=1, keepdims=True)
    cand_j = rc * LC + lax.broadcasted_iota(jnp.int32, (BM, LC), 1)
    rarg = jnp.min(jnp.where(rv == rmax, cand_j, 2**30),
                   axis=1, keepdims=True)
    nn12_ref[...] = rarg
    sc_ref[...] = rmax

    SC_ = 8
    cv = sim[0:SC_, :]
    cc = jnp.zeros((SC_, N2), jnp.int32)
    for c in range(1, BM // SC_):
        v = sim[c * SC_:(c + 1) * SC_, :]
        m = v > cv
        cv = jnp.where(m, v, cv)
        cc = jnp.where(m, c, cc)
    bmax = jnp.max(cv, axis=0, keepdims=True)
    cand_i = cc * SC_ + lax.broadcasted_iota(jnp.int32, (SC_, N2), 0) + i * BM
    barg = jnp.min(jnp.where(cv == bmax, cand_i, 2**30),
                   axis=0, keepdims=True)

    @pl.when(i == 0)
    def _init():
        cmax_ref[...] = bmax
        nn21_ref[...] = barg

    @pl.when(i > 0)
    def _update():
        prev_max = cmax_ref[...]
        prev_arg = nn21_ref[...]
        better = bmax > prev_max
        cmax_ref[...] = jnp.where(better, bmax, prev_max)
        nn21_ref[...] = jnp.where(better, barg, prev_arg)


def _stage1(d0, d1):
    grid = (N1 // BM,)
    return pl.pallas_call(
        _stage1_body,
        grid=grid,
        in_specs=[
            pl.BlockSpec((D, BM), lambda i: (0, i)),
            pl.BlockSpec((D, N2), lambda i: (0, 0)),
        ],
        out_specs=[
            pl.BlockSpec((BM, 1), lambda i: (i, 0)),
            pl.BlockSpec((BM, 1), lambda i: (i, 0)),
            pl.BlockSpec((1, N2), lambda i: (0, 0)),
        ],
        out_shape=[
            jax.ShapeDtypeStruct((N1, 1), jnp.int32),
            jax.ShapeDtypeStruct((N1, 1), jnp.float32),
            jax.ShapeDtypeStruct((1, N2), jnp.int32),
        ],
        scratch_shapes=[pltpu.VMEM((1, N2), jnp.float32)],
    )(d0, d1)


def _stage2_body(nn12_hbm, nn21_hbm, sc_hbm, oi_hbm, os_hbm,
                 idx_v, g_v, s_v, oi_v, os_v, sem):
    c = lax.axis_index("c")
    s = lax.axis_index("s")
    wid = s * NC + c
    base = wid * CHUNK
    pltpu.sync_copy(nn12_hbm.at[pl.ds(base, CHUNK)], idx_v)
    pltpu.sync_copy(sc_hbm.at[pl.ds(base, CHUNK)], s_v)
    pltpu.async_copy(nn21_hbm.at[idx_v], g_v, sem).wait()
    for k in range(CHUNK // L):
        idx = idx_v[pl.ds(k * L, L)]
        g = g_v[pl.ds(k * L, L)]
        ids = lax.iota(jnp.int32, L) + (base + k * L)
        m = g == ids
        oi_v[pl.ds(k * L, L)] = jnp.where(m, idx, jnp.int32(-1))
        os_v[pl.ds(k * L, L)] = jnp.where(m, s_v[pl.ds(k * L, L)],
                                          jnp.float32(-1.0))
    pltpu.sync_copy(oi_v, oi_hbm.at[pl.ds(base, CHUNK)])
    pltpu.sync_copy(os_v, os_hbm.at[pl.ds(base, CHUNK)])


@functools.lru_cache(maxsize=1)
def _get_stage2():
    return functools.partial(
        pl.kernel,
        mesh=plsc.VectorSubcoreMesh(core_axis_name="c", subcore_axis_name="s"),
        out_type=[
            jax.ShapeDtypeStruct((N1,), jnp.int32),
            jax.ShapeDtypeStruct((N1,), jnp.float32),
        ],
        scratch_types=[
            pltpu.VMEM((CHUNK,), jnp.int32),
            pltpu.VMEM((CHUNK,), jnp.int32),
            pltpu.VMEM((CHUNK,), jnp.float32),
            pltpu.VMEM((CHUNK,), jnp.int32),
            pltpu.VMEM((CHUNK,), jnp.float32),
            pltpu.SemaphoreType.DMA,
        ],
    )(_stage2_body)


def kernel(descriptors0, descriptors1, keypoints0, keypoints1):
    d0 = jnp.squeeze(descriptors0, axis=0)
    d1 = jnp.squeeze(descriptors1, axis=0)
    nn12, scores, nn21 = _stage1(d0, d1)
    oi, os = _get_stage2()(nn12.reshape(N1), nn21.reshape(N2),
                           scores.reshape(N1))
    indices0 = oi[None, :]
    mscores0 = os[None, :]
    return indices0, indices0, mscores0, mscores0


# speedup vs baseline: 2.6197x; 1.2451x over previous
import functools

import jax
import jax.numpy as jnp
from jax import lax
from jax.experimental import pallas as pl
from jax.experimental.pallas import tpu as pltpu
from jax.experimental.pallas import tpu_sc as plsc

N1 = 4096
N2 = 4096
D = 256
BM = 256

NC = 2
NS = 16
NW = NC * NS
CHUNK = N1 // NW
L = 16

def _stage1_body(d0_ref, d1_ref, nn12_ref, sc_ref, nn21_ref, cmax_ref):
    i = pl.program_id(0)
    sim = jax.lax.dot_general(
        d0_ref[...], d1_ref[...],
        dimension_numbers=(((0,), (0,)), ((), ())),
        preferred_element_type=jnp.float32,
    )

    LC = 128
    rv = sim[:, 0:LC]
    rc = jnp.zeros((BM, LC), jnp.int32)
    for c in range(1, N2 // LC):
        v = sim[:, c * LC:(c + 1) * LC]
        m = v > rv
        rv = jnp.where(m, v, rv)
        rc = jnp.where(m, c, rc)
    rmax = jnp.max(rv, axis=1, keepdims=True)
    cand_j = rc * LC + lax.broadcasted_iota(jnp.int32, (BM, LC), 1)
    rarg = jnp.min(jnp.where(rv == rmax, cand_j, 2**30),
                   axis=1, keepdims=True)
    nn12_ref[...] = rarg
    sc_ref[...] = rmax

    SC_ = 8
    cv = sim[0:SC_, :]
    cc = jnp.zeros((SC_, N2), jnp.int32)
    for c in range(1, BM // SC_):
        v = sim[c * SC_:(c + 1) * SC_, :]
        m = v > cv
        cv = jnp.where(m, v, cv)
        cc = jnp.where(m, c, cc)
    bmax = jnp.max(cv, axis=0, keepdims=True)
    cand_i = cc * SC_ + lax.broadcasted_iota(jnp.int32, (SC_, N2), 0) + i * BM
    barg = jnp.min(jnp.where(cv == bmax, cand_i, 2**30),
                   axis=0, keepdims=True)

    @pl.when(i == 0)
    def _init():
        cmax_ref[...] = bmax
        nn21_ref[...] = barg

    @pl.when(i > 0)
    def _update():
        prev_max = cmax_ref[...]
        prev_arg = nn21_ref[...]
        better = bmax > prev_max
        cmax_ref[...] = jnp.where(better, bmax, prev_max)
        nn21_ref[...] = jnp.where(better, barg, prev_arg)


def _stage1(d0, d1):
    grid = (N1 // BM,)
    return pl.pallas_call(
        _stage1_body,
        grid=grid,
        in_specs=[
            pl.BlockSpec((D, BM), lambda i: (0, i)),
            pl.BlockSpec((D, N2), lambda i: (0, 0)),
        ],
        out_specs=[
            pl.BlockSpec((BM, 1), lambda i: (i, 0)),
            pl.BlockSpec((BM, 1), lambda i: (i, 0)),
            pl.BlockSpec((1, N2), lambda i: (0, 0)),
        ],
        out_shape=[
            jax.ShapeDtypeStruct((N1, 1), jnp.int32),
            jax.ShapeDtypeStruct((N1, 1), jnp.float32),
            jax.ShapeDtypeStruct((1, N2), jnp.int32),
        ],
        scratch_shapes=[pltpu.VMEM((1, N2), jnp.float32)],
    )(d0, d1)


BM2 = 512


def _stage2_tc_body(nn12_ref, sc_ref, carg_ref, oi_ref, os_ref):
    i = pl.program_id(0)
    nn12b = nn12_ref[...]
    carg_row = carg_ref[...]
    lane = lax.broadcasted_iota(jnp.int32, (BM2, N2), 1)
    eq = lane == nn12b
    g = jnp.sum(jnp.where(eq, carg_row, 0), axis=1, keepdims=True)
    ids = lax.broadcasted_iota(jnp.int32, (BM2, 1), 0) + i * BM2
    m = g == ids
    oi_ref[...] = jnp.where(m, nn12b, jnp.int32(-1))
    os_ref[...] = jnp.where(m, sc_ref[...], jnp.float32(-1.0))


def _stage2_tc(nn12, scores, nn21):
    return pl.pallas_call(
        _stage2_tc_body,
        grid=(N1 // BM2,),
        in_specs=[
            pl.BlockSpec((BM2, 1), lambda i: (i, 0)),
            pl.BlockSpec((BM2, 1), lambda i: (i, 0)),
            pl.BlockSpec((1, N2), lambda i: (0, 0)),
        ],
        out_specs=[
            pl.BlockSpec((BM2, 1), lambda i: (i, 0)),
            pl.BlockSpec((BM2, 1), lambda i: (i, 0)),
        ],
        out_shape=[
            jax.ShapeDtypeStruct((N1, 1), jnp.int32),
            jax.ShapeDtypeStruct((N1, 1), jnp.float32),
        ],
    )(nn12, scores, nn21)


def _stage2_body(nn12_hbm, nn21_hbm, sc_hbm, oi_hbm, os_hbm,
                 idx_v, g_v, s_v, oi_v, os_v, sem):
    c = lax.axis_index("c")
    s = lax.axis_index("s")
    wid = s * NC + c
    base = wid * CHUNK
    pltpu.sync_copy(nn12_hbm.at[pl.ds(base, CHUNK)], idx_v)
    pltpu.sync_copy(sc_hbm.at[pl.ds(base, CHUNK)], s_v)
    pltpu.async_copy(nn21_hbm.at[idx_v], g_v, sem).wait()
    for k in range(CHUNK // L):
        idx = idx_v[pl.ds(k * L, L)]
        g = g_v[pl.ds(k * L, L)]
        ids = lax.iota(jnp.int32, L) + (base + k * L)
        m = g == ids
        oi_v[pl.ds(k * L, L)] = jnp.where(m, idx, jnp.int32(-1))
        os_v[pl.ds(k * L, L)] = jnp.where(m, s_v[pl.ds(k * L, L)],
                                          jnp.float32(-1.0))
    pltpu.sync_copy(oi_v, oi_hbm.at[pl.ds(base, CHUNK)])
    pltpu.sync_copy(os_v, os_hbm.at[pl.ds(base, CHUNK)])


@functools.lru_cache(maxsize=1)
def _get_stage2():
    return functools.partial(
        pl.kernel,
        mesh=plsc.VectorSubcoreMesh(core_axis_name="c", subcore_axis_name="s"),
        out_type=[
            jax.ShapeDtypeStruct((N1,), jnp.int32),
            jax.ShapeDtypeStruct((N1,), jnp.float32),
        ],
        scratch_types=[
            pltpu.VMEM((CHUNK,), jnp.int32),
            pltpu.VMEM((CHUNK,), jnp.int32),
            pltpu.VMEM((CHUNK,), jnp.float32),
            pltpu.VMEM((CHUNK,), jnp.int32),
            pltpu.VMEM((CHUNK,), jnp.float32),
            pltpu.SemaphoreType.DMA,
        ],
    )(_stage2_body)


def kernel(descriptors0, descriptors1, keypoints0, keypoints1):
    d0 = jnp.squeeze(descriptors0, axis=0)
    d1 = jnp.squeeze(descriptors1, axis=0)
    nn12, scores, nn21 = _stage1(d0, d1)
    oi, os = _stage2_tc(nn12, scores, nn21)
    indices0 = oi.reshape(1, N1)
    mscores0 = os.reshape(1, N1)
    return indices0, indices0, mscores0, mscores0


# speedup vs baseline: 3.5879x; 1.3696x over previous
import jax
import jax.numpy as jnp
from jax import lax
from jax.experimental import pallas as pl
from jax.experimental.pallas import tpu as pltpu

N1 = 4096
N2 = 4096
D = 256
BM = 512
NSTEP = N1 // BM


def _body(d0_ref, d1_ref, oi_ref, os_ref, nn12_s, sc_s, cmax_s, carg_s):
    i = pl.program_id(0)
    sim = jax.lax.dot_general(
        d0_ref[...], d1_ref[...],
        dimension_numbers=(((0,), (0,)), ((), ())),
        preferred_element_type=jnp.float32,
    )

    LC = 128
    rv = sim[:, 0:LC]
    rc = jnp.zeros((BM, LC), jnp.int32)
    for c in range(1, N2 // LC):
        v = sim[:, c * LC:(c + 1) * LC]
        m = v > rv
        rv = jnp.where(m, v, rv)
        rc = jnp.where(m, c, rc)
    rmax = jnp.max(rv, axis=1, keepdims=True)
    cand_j = rc * LC + lax.broadcasted_iota(jnp.int32, (BM, LC), 1)
    rarg = jnp.min(jnp.where(rv == rmax, cand_j, 2**30),
                   axis=1, keepdims=True)
    nn12_s[pl.ds(i * BM, BM), :] = rarg
    sc_s[pl.ds(i * BM, BM), :] = rmax

    SC_ = 8
    cv = sim[0:SC_, :]
    cc = jnp.zeros((SC_, N2), jnp.int32)
    for c in range(1, BM // SC_):
        v = sim[c * SC_:(c + 1) * SC_, :]
        m = v > cv
        cv = jnp.where(m, v, cv)
        cc = jnp.where(m, c, cc)
    bmax = jnp.max(cv, axis=0, keepdims=True)
    cand_i = cc * SC_ + lax.broadcasted_iota(jnp.int32, (SC_, N2), 0) + i * BM
    barg = jnp.min(jnp.where(cv == bmax, cand_i, 2**30),
                   axis=0, keepdims=True)

    @pl.when(i == 0)
    def _init():
        cmax_s[...] = bmax
        carg_s[...] = barg

    @pl.when(i > 0)
    def _update():
        prev_max = cmax_s[...]
        prev_arg = carg_s[...]
        better = bmax > prev_max
        cmax_s[...] = jnp.where(better, bmax, prev_max)
        carg_s[...] = jnp.where(better, barg, prev_arg)

    @pl.when(i == NSTEP - 1)
    def _final():
        nn12 = nn12_s[...]
        carg = carg_s[...]
        tbl = jnp.reshape(carg.astype(jnp.float32), (32, 128))
        hi = nn12 // 128
        lo = nn12 - hi * 128
        oh = (lax.broadcasted_iota(jnp.int32, (N1, 32), 1) == hi
              ).astype(jnp.float32)
        s = jax.lax.dot_general(
            oh, tbl, dimension_numbers=(((1,), (0,)), ((), ())),
            preferred_element_type=jnp.float32,
        )
        lane = lax.broadcasted_iota(jnp.int32, (N1, 128), 1)
        g = jnp.sum(jnp.where(lane == lo, s, 0.0), axis=1, keepdims=True)
        ids = lax.broadcasted_iota(jnp.int32, (N1, 1), 0)
        mut = g == ids.astype(jnp.float32)
        oi_ref[...] = jnp.where(mut, nn12, jnp.int32(-1))
        os_ref[...] = jnp.where(mut, sc_s[...], jnp.float32(-1.0))


def _run(d0, d1):
    return pl.pallas_call(
        _body,
        grid=(NSTEP,),
        in_specs=[
            pl.BlockSpec((D, BM), lambda i: (0, i)),
            pl.BlockSpec((D, N2), lambda i: (0, 0)),
        ],
        out_specs=[
            pl.BlockSpec((N1, 1), lambda i: (0, 0)),
            pl.BlockSpec((N1, 1), lambda i: (0, 0)),
        ],
        out_shape=[
            jax.ShapeDtypeStruct((N1, 1), jnp.int32),
            jax.ShapeDtypeStruct((N1, 1), jnp.float32),
        ],
        scratch_shapes=[
            pltpu.VMEM((N1, 1), jnp.int32),
            pltpu.VMEM((N1, 1), jnp.float32),
            pltpu.VMEM((1, N2), jnp.float32),
            pltpu.VMEM((1, N2), jnp.int32),
        ],
    )(d0, d1)


def kernel(descriptors0, descriptors1, keypoints0, keypoints1):
    d0 = jnp.squeeze(descriptors0, axis=0)
    d1 = jnp.squeeze(descriptors1, axis=0)
    oi, os = _run(d0, d1)
    indices0 = oi.reshape(1, N1)
    mscores0 = os.reshape(1, N1)
    return indices0, indices0, mscores0, mscores0
